# Initial kernel scaffold; baseline (speedup 1.0000x reference)
import jax
import jax.numpy as jnp
from jax.experimental import pallas as pl


def kernel(x, coefficients_vect):
    raise NotImplementedError("write your pallas kernel here")


# speedup vs baseline: 421.1816x; 421.1816x over previous
import functools

import jax
import jax.numpy as jnp
from jax import lax
from jax.experimental import pallas as pl
from jax.experimental.pallas import tpu as pltpu
from jax.experimental.pallas import tpu_sc as plsc

_SIZE = 4096
_NC = 2
_NS = 16
_NW = _NC * _NS
_L = 16
_CH = 16384


@functools.partial(jax.jit, static_argnums=(3,))
def _run(x_flat, lo_t, d_t, n):
    per_w = n // _NW
    nch = per_w // _CH
    mesh = plsc.VectorSubcoreMesh(core_axis_name="c", subcore_axis_name="s")

    @functools.partial(
        pl.kernel,
        out_type=jax.ShapeDtypeStruct((n,), jnp.float32),
        mesh=mesh,
        scratch_types=[
            pltpu.VMEM((_SIZE,), jnp.float32),
            pltpu.VMEM((_SIZE,), jnp.float32),
            pltpu.VMEM((_CH,), jnp.float32),
            pltpu.VMEM((_CH,), jnp.float32),
        ],
        compiler_params=pltpu.CompilerParams(needs_layout_passes=False),
    )
    def k(x_hbm, lo_hbm, d_hbm, out_hbm, lo_v, d_v, in_v, out_v):
        wid = lax.axis_index("s") * _NC + lax.axis_index("c")
        base = wid * per_w
        pltpu.sync_copy(lo_hbm, lo_v)
        pltpu.sync_copy(d_hbm, d_v)

        def chunk_body(g, carry):
            off = base + g * _CH
            pltpu.sync_copy(x_hbm.at[pl.ds(off, _CH)], in_v)

            def vec_body(j, c2):
                xv = in_v[pl.ds(j * _L, _L)]
                t = xv * 1000.0 + 2048.0
                t = jnp.minimum(jnp.maximum(t, 0.0), 4095.0)
                i = t.astype(jnp.int32)
                f = t - i.astype(jnp.float32)
                lo = plsc.load_gather(lo_v, [i])
                dd = plsc.load_gather(d_v, [i])
                out_v[pl.ds(j * _L, _L)] = lo + f * dd
                return c2

            lax.fori_loop(0, _CH // _L, vec_body, 0, unroll=4)
            pltpu.sync_copy(out_v, out_hbm.at[pl.ds(off, _CH)])
            return carry

        lax.fori_loop(0, nch, chunk_body, 0)

    return k(x_flat, lo_t, d_t)


def kernel(x, coefficients_vect):
    c = coefficients_vect
    lo_t = jnp.roll(c, 2048)
    d_t = jnp.roll(c, 2047) - lo_t
    x_flat = x.reshape(-1)
    out = _run(x_flat, lo_t, d_t, x_flat.shape[0])
    return out.reshape(x.shape)


# speedup vs baseline: 436.2439x; 1.0358x over previous
import functools

import jax
import jax.numpy as jnp
from jax import lax
from jax.experimental import pallas as pl
from jax.experimental.pallas import tpu as pltpu
from jax.experimental.pallas import tpu_sc as plsc

_SIZE = 4096
_NC = 2
_NS = 16
_NW = _NC * _NS
_L = 16
_CH = 16384


@functools.partial(jax.jit, static_argnums=(3,))
def _run(x_flat, lo_t, d_t, n):
    per_w = n // _NW
    nch = per_w // _CH
    mesh = plsc.VectorSubcoreMesh(core_axis_name="c", subcore_axis_name="s")

    @functools.partial(
        pl.kernel,
        out_type=jax.ShapeDtypeStruct((n,), jnp.float32),
        mesh=mesh,
        scratch_types=[
            pltpu.VMEM((_SIZE,), jnp.float32),
            pltpu.VMEM((_SIZE,), jnp.float32),
            pltpu.VMEM((2, _CH), jnp.float32),
            pltpu.VMEM((2, _CH), jnp.float32),
            pltpu.SemaphoreType.DMA,
            pltpu.SemaphoreType.DMA,
            pltpu.SemaphoreType.DMA,
            pltpu.SemaphoreType.DMA,
        ],
        compiler_params=pltpu.CompilerParams(needs_layout_passes=False),
    )
    def k(x_hbm, lo_hbm, d_hbm, out_hbm, lo_v, d_v, in_v, out_v,
          sem_in0, sem_in1, sem_out0, sem_out1):
        wid = lax.axis_index("s") * _NC + lax.axis_index("c")
        base = wid * per_w
        sems_in = (sem_in0, sem_in1)
        sems_out = (sem_out0, sem_out1)

        pltpu.sync_copy(lo_hbm, lo_v)
        pltpu.sync_copy(d_hbm, d_v)

        def in_cp(g, b):
            return pltpu.make_async_copy(
                x_hbm.at[pl.ds(base + g * _CH, _CH)], in_v.at[b], sems_in[b])

        def out_cp(g, b):
            return pltpu.make_async_copy(
                out_v.at[b], out_hbm.at[pl.ds(base + g * _CH, _CH)],
                sems_out[b])

        def compute(b):
            def _vec(jj, carry2):
                j = jj * _L
                xv = in_v[b, pl.ds(j, _L)]
                t = xv * 1000.0 + 2048.0
                t = jnp.minimum(jnp.maximum(t, 0.0), 4095.0)
                i = t.astype(jnp.int32)
                f = t - i.astype(jnp.float32)
                lo = plsc.load_gather(lo_v, [i])
                dd = plsc.load_gather(d_v, [i])
                out_v[b, pl.ds(j, _L)] = lo + f * dd
                return carry2

            lax.fori_loop(0, _CH // _L, _vec, 0, unroll=8)

        in_cp(0, 0).start()
        in_cp(1, 1).start()

        def pair(p, carry):
            for b in range(2):
                g = p * 2 + b
                in_cp(g, b).wait()

                @pl.when(g >= 2)
                def _wait_out():
                    out_cp(g - 2, b).wait()

                compute(b)
                out_cp(g, b).start()

                @pl.when(g + 2 < nch)
                def _next_in():
                    in_cp(g + 2, b).start()

            return carry

        lax.fori_loop(0, nch // 2, pair, 0)
        out_cp(nch - 2, 0).wait()
        out_cp(nch - 1, 1).wait()

    return k(x_flat, lo_t, d_t)


def kernel(x, coefficients_vect):
    c = coefficients_vect
    lo_t = jnp.roll(c, 2048)
    d_t = jnp.roll(c, 2047) - lo_t
    x_flat = x.reshape(-1)
    out = _run(x_flat, lo_t, d_t, x_flat.shape[0])
    return out.reshape(x.shape)


# speedup vs baseline: 1316.7320x; 3.0183x over previous
import functools

import jax
import jax.numpy as jnp
from jax import lax
from jax.experimental import pallas as pl
from jax.experimental.pallas import tpu as pltpu
from jax.experimental.pallas import tpu_sc as plsc

_SIZE = 4096
_NC = 2
_NS = 16
_NW = _NC * _NS
_L = 16
_CH = 16384


@functools.partial(jax.jit, static_argnums=(3,))
def _run(x_flat, lo_t, d_t, n):
    per_w = n // _NW
    nch = per_w // _CH
    mesh = plsc.VectorSubcoreMesh(core_axis_name="c", subcore_axis_name="s")

    @functools.partial(
        pl.kernel,
        out_type=jax.ShapeDtypeStruct((n,), jnp.float32),
        mesh=mesh,
        scratch_types=[
            pltpu.VMEM((_SIZE,), jnp.float32),
            pltpu.VMEM((_SIZE,), jnp.float32),
            pltpu.VMEM((2, _CH), jnp.float32),
            pltpu.VMEM((2, _CH), jnp.float32),
            pltpu.SemaphoreType.DMA,
            pltpu.SemaphoreType.DMA,
            pltpu.SemaphoreType.DMA,
            pltpu.SemaphoreType.DMA,
        ],
        compiler_params=pltpu.CompilerParams(needs_layout_passes=False),
    )
    def k(x_hbm, lo_hbm, d_hbm, out_hbm, lo_v, d_v, in_v, out_v,
          sem_in0, sem_in1, sem_out0, sem_out1):
        wid = lax.axis_index("s") * _NC + lax.axis_index("c")
        base = wid * per_w
        sems_in = (sem_in0, sem_in1)
        sems_out = (sem_out0, sem_out1)

        pltpu.sync_copy(lo_hbm, lo_v)
        pltpu.sync_copy(d_hbm, d_v)

        def in_cp(g, b):
            return pltpu.make_async_copy(
                x_hbm.at[pl.ds(base + g * _CH, _CH)], in_v.at[b], sems_in[b])

        def out_cp(g, b):
            return pltpu.make_async_copy(
                out_v.at[b], out_hbm.at[pl.ds(base + g * _CH, _CH)],
                sems_out[b])

        def compute(b):
            @plsc.parallel_loop(0, _CH, step=_L, unroll=8)
            def _vec(j):
                xv = in_v[b, pl.ds(j, _L)]
                t = xv * 1000.0 + 2048.0
                t = jnp.minimum(jnp.maximum(t, 0.0), 4095.0)
                i = t.astype(jnp.int32)
                f = t - i.astype(jnp.float32)
                lo = plsc.load_gather(lo_v, [i])
                dd = plsc.load_gather(d_v, [i])
                out_v[b, pl.ds(j, _L)] = lo + f * dd

        in_cp(0, 0).start()
        in_cp(1, 1).start()

        def pair(p, carry):
            for b in range(2):
                g = p * 2 + b
                in_cp(g, b).wait()

                @pl.when(g >= 2)
                def _wait_out():
                    out_cp(g - 2, b).wait()

                compute(b)
                out_cp(g, b).start()

                @pl.when(g + 2 < nch)
                def _next_in():
                    in_cp(g + 2, b).start()

            return carry

        lax.fori_loop(0, nch // 2, pair, 0)
        out_cp(nch - 2, 0).wait()
        out_cp(nch - 1, 1).wait()

    return k(x_flat, lo_t, d_t)


def kernel(x, coefficients_vect):
    c = coefficients_vect
    lo_t = jnp.roll(c, 2048)
    d_t = jnp.roll(c, 2047) - lo_t
    x_flat = x.reshape(-1)
    out = _run(x_flat, lo_t, d_t, x_flat.shape[0])
    return out.reshape(x.shape)
